# Initial kernel scaffold; baseline (speedup 1.0000x reference)
import jax
import jax.numpy as jnp
from jax.experimental import pallas as pl


def kernel(x, Wg, W1, b1, W2, b2):
    raise NotImplementedError("write your pallas kernel here")


# speedup vs baseline: 3.4132x; 3.4132x over previous
import jax
import jax.numpy as jnp
from jax.experimental import pallas as pl
from jax.experimental.pallas import tpu as pltpu

EMBED_DIM = 64
FFN_DIM = 128
NUM_EXPERTS = 4


def _moe_kernel(x_ref, wg_ref, w1_ref, b1_ref, w2_ref, b2_ref, o_ref):
    xb = x_ref[:]
    logits = jax.lax.dot_general(
        xb, wg_ref[:], (((1,), (0,)), ((), ())),
        preferred_element_type=jnp.float32)

    e_iota = jax.lax.broadcasted_iota(jnp.int32, logits.shape, 1)
    m1 = jnp.max(logits, axis=-1, keepdims=True)
    idx1 = jnp.min(jnp.where(logits == m1, e_iota, NUM_EXPERTS),
                   axis=-1, keepdims=True)
    masked = jnp.where(e_iota == idx1, -jnp.inf, logits)
    m2 = jnp.max(masked, axis=-1, keepdims=True)
    idx2 = jnp.min(jnp.where(masked == m2, e_iota, NUM_EXPERTS),
                   axis=-1, keepdims=True)
    p1 = 1.0 / (1.0 + jnp.exp(m2 - m1))
    p2 = 1.0 - p1
    gate_w = (jnp.where(e_iota == idx1, p1, 0.0)
              + jnp.where(e_iota == idx2, p2, 0.0))

    h = jax.lax.dot_general(
        xb, w1_ref[:], (((1,), (0,)), ((), ())),
        preferred_element_type=jnp.float32) + b1_ref[:]
    h = jnp.maximum(h, 0.0)

    col_e = jax.lax.broadcasted_iota(jnp.int32, h.shape, 1) // FFN_DIM
    wcol = (jnp.where(col_e == idx1, p1, 0.0)
            + jnp.where(col_e == idx2, p2, 0.0))
    hw = h * wcol

    out = jax.lax.dot_general(
        hw, w2_ref[:], (((1,), (0,)), ((), ())),
        preferred_element_type=jnp.float32)
    out = out + jax.lax.dot_general(
        gate_w, b2_ref[:], (((1,), (0,)), ((), ())),
        preferred_element_type=jnp.float32)
    o_ref[:] = out


def kernel(x, Wg, W1, b1, W2, b2):
    x = x.reshape(-1, x.shape[-1])
    T, D = x.shape
    E, _, F = W1.shape
    w1_cat = W1.transpose(1, 0, 2).reshape(D, E * F)
    b1_cat = b1.reshape(1, E * F)
    w2_cat = W2.reshape(E * F, D)

    TB = 1024
    grid = (T // TB,)
    out = pl.pallas_call(
        _moe_kernel,
        grid=grid,
        in_specs=[
            pl.BlockSpec((TB, D), lambda i: (i, 0)),
            pl.BlockSpec((D, E), lambda i: (0, 0)),
            pl.BlockSpec((D, E * F), lambda i: (0, 0)),
            pl.BlockSpec((1, E * F), lambda i: (0, 0)),
            pl.BlockSpec((E * F, D), lambda i: (0, 0)),
            pl.BlockSpec((E, D), lambda i: (0, 0)),
        ],
        out_specs=pl.BlockSpec((TB, D), lambda i: (i, 0)),
        out_shape=jax.ShapeDtypeStruct((T, D), jnp.float32),
        compiler_params=pltpu.CompilerParams(
            dimension_semantics=("arbitrary",)),
    )(x, Wg, w1_cat, b1_cat, w2_cat, b2)
    return out


# speedup vs baseline: 4.4327x; 1.2987x over previous
import jax
import jax.numpy as jnp
from jax.experimental import pallas as pl
from jax.experimental.pallas import tpu as pltpu

EMBED_DIM = 64
FFN_DIM = 128
NUM_EXPERTS = 4


def _moe_kernel(x_ref, wg_ref, w1_ref, b1_ref, w2_ref, b2_ref, ex_ref, o_ref):
    xb = x_ref[:]
    lT = jax.lax.dot_general(
        wg_ref[:], xb, (((1,), (1,)), ((), ())),
        preferred_element_type=jnp.float32)

    e_iota = jax.lax.broadcasted_iota(jnp.int32, lT.shape, 0)
    m1 = jnp.max(lT, axis=0, keepdims=True)
    idx1 = jnp.min(jnp.where(lT == m1, e_iota, NUM_EXPERTS),
                   axis=0, keepdims=True)
    masked = jnp.where(e_iota == idx1, -jnp.inf, lT)
    m2 = jnp.max(masked, axis=0, keepdims=True)
    idx2 = jnp.min(jnp.where(masked == m2, e_iota, NUM_EXPERTS),
                   axis=0, keepdims=True)
    p1 = 1.0 / (1.0 + jnp.exp(m2 - m1))
    p2 = 1.0 - p1
    wT = (jnp.where(e_iota == idx1, p1, 0.0)
          + jnp.where(e_iota == idx2, p2, 0.0))

    h = jax.lax.dot_general(
        xb, w1_ref[:], (((1,), (0,)), ((), ())),
        preferred_element_type=jnp.float32) + b1_ref[:]
    h = jnp.maximum(h, 0.0)

    wcol = jax.lax.dot_general(
        wT, ex_ref[:], (((0,), (0,)), ((), ())),
        preferred_element_type=jnp.float32)
    hw = h * wcol

    out = jax.lax.dot_general(
        hw, w2_ref[:], (((1,), (0,)), ((), ())),
        preferred_element_type=jnp.float32)
    out = out + jax.lax.dot_general(
        wT, b2_ref[:], (((0,), (0,)), ((), ())),
        preferred_element_type=jnp.float32)
    o_ref[:] = out


def kernel(x, Wg, W1, b1, W2, b2):
    x = x.reshape(-1, x.shape[-1])
    T, D = x.shape
    E, _, F = W1.shape
    w1_cat = W1.transpose(1, 0, 2).reshape(D, E * F)
    b1_cat = b1.reshape(1, E * F)
    w2_cat = W2.reshape(E * F, D)
    expand = jnp.repeat(jnp.eye(E, dtype=jnp.float32), F, axis=1)

    TB = 1024
    grid = (T // TB,)
    out = pl.pallas_call(
        _moe_kernel,
        grid=grid,
        in_specs=[
            pl.BlockSpec((TB, D), lambda i: (i, 0)),
            pl.BlockSpec((E, D), lambda i: (0, 0)),
            pl.BlockSpec((D, E * F), lambda i: (0, 0)),
            pl.BlockSpec((1, E * F), lambda i: (0, 0)),
            pl.BlockSpec((E * F, D), lambda i: (0, 0)),
            pl.BlockSpec((E, D), lambda i: (0, 0)),
            pl.BlockSpec((E, E * F), lambda i: (0, 0)),
        ],
        out_specs=pl.BlockSpec((TB, D), lambda i: (i, 0)),
        out_shape=jax.ShapeDtypeStruct((T, D), jnp.float32),
        compiler_params=pltpu.CompilerParams(
            dimension_semantics=("arbitrary",)),
    )(x, Wg.T, w1_cat, b1_cat, w2_cat, b2, expand)
    return out
